# Initial kernel scaffold; baseline (speedup 1.0000x reference)
import jax
import jax.numpy as jnp
from jax.experimental import pallas as pl


def kernel(x, norm_w, Win, conv_w, conv_b, Wx, Wdt, bdt, A_log, Dp, Wblk, Wout, bout):
    raise NotImplementedError("write your pallas kernel here")


# speedup vs baseline: 11.3882x; 11.3882x over previous
import jax
import jax.numpy as jnp
from jax.experimental import pallas as pl
from jax.experimental.pallas import tpu as pltpu

_INTERPRET = False

B, L, D = 2, 1024, 1024
DI = 2048
NS = 16
DTR = 64
KC = 4
EPS = 1e-5

TDI = 256
NB = DI // TDI


def _silu(v):
    return v * jax.nn.sigmoid(v)


def _proj_kernel(x_ref, nw_ref, win_ref, cw_ref, cb_ref, o_ref):
    j = pl.program_id(1)
    x = x_ref[0]
    ssq = jnp.mean(x * x, axis=-1, keepdims=True)
    xn = x * jax.lax.rsqrt(ssq + EPS) * nw_ref[...]
    xz = jnp.dot(xn, win_ref[...], preferred_element_type=jnp.float32)
    w = cw_ref[...]
    acc = xz * w[KC - 1][None, :]
    for k in range(KC - 1):
        sh = KC - 1 - k
        shifted = jnp.concatenate(
            [jnp.zeros((sh, DI), jnp.float32), xz[: L - sh]], axis=0)
        acc = acc + shifted * w[k][None, :]
    acc = acc + cb_ref[...]
    conv_out = _silu(acc)
    o_ref[0, 0] = jnp.where(j == 0, conv_out, xz)


def _proj(x, nw, win, cwT, cb):
    return pl.pallas_call(
        _proj_kernel,
        out_shape=jax.ShapeDtypeStruct((B, 2, L, DI), jnp.float32),
        grid=(B, 2),
        in_specs=[
            pl.BlockSpec((1, L, D), lambda b, j: (b, 0, 0)),
            pl.BlockSpec((1, D), lambda b, j: (0, 0)),
            pl.BlockSpec((D, DI), lambda b, j: (0, j)),
            pl.BlockSpec((KC, DI), lambda b, j: (0, 0)),
            pl.BlockSpec((1, DI), lambda b, j: (0, 0)),
        ],
        out_specs=pl.BlockSpec((1, 1, L, DI), lambda b, j: (b, j, 0, 0)),
        compiler_params=pltpu.CompilerParams(
            dimension_semantics=("parallel", "arbitrary"),
            vmem_limit_bytes=56 * 1024 * 1024,
        ),
        name="mamba_proj",
        interpret=_INTERPRET,
    )(x, nw, win, cwT, cb)


_MT = 256


def _dbc_kernel(xc_ref, wx_ref, wdt_ref, bdt_ref, d_ref, b_ref, c_ref):
    xc = xc_ref[...]
    dbc = jnp.dot(xc, wx_ref[...], preferred_element_type=jnp.float32)
    dt = dbc[:, :DTR]
    d_ref[...] = jax.nn.softplus(
        jnp.dot(dt, wdt_ref[...], preferred_element_type=jnp.float32)
        + bdt_ref[...])
    b_ref[...] = dbc[:, DTR:DTR + NS]
    c_ref[...] = dbc[:, DTR + NS:DTR + 2 * NS]


def _dbc(xc2, wx, wdt, bdt2):
    M = B * L
    grid = (M // _MT,)
    return pl.pallas_call(
        _dbc_kernel,
        out_shape=(
            jax.ShapeDtypeStruct((M, DI), jnp.float32),
            jax.ShapeDtypeStruct((M, NS), jnp.float32),
            jax.ShapeDtypeStruct((M, NS), jnp.float32),
        ),
        grid=grid,
        in_specs=[
            pl.BlockSpec((_MT, DI), lambda i: (i, 0)),
            pl.BlockSpec((DI, DTR + 2 * NS), lambda i: (0, 0)),
            pl.BlockSpec((DTR, DI), lambda i: (0, 0)),
            pl.BlockSpec((1, DI), lambda i: (0, 0)),
        ],
        out_specs=(
            pl.BlockSpec((_MT, DI), lambda i: (i, 0)),
            pl.BlockSpec((_MT, NS), lambda i: (i, 0)),
            pl.BlockSpec((_MT, NS), lambda i: (i, 0)),
        ),
        compiler_params=pltpu.CompilerParams(
            dimension_semantics=("parallel",),
            vmem_limit_bytes=48 * 1024 * 1024,
        ),
        name="mamba_dbc",
        interpret=_INTERPRET,
    )(xc2, wx, wdt, bdt2)


_PC = 128


def _scan_kernel(u_ref, d_ref, bm_ref, cm_ref, z_ref, at_ref, dp_ref,
                 o_ref, dA_s, dBu_s):
    A = -jnp.exp(at_ref[...])

    def pre(i, _):
        sl = pl.ds(i * _PC, _PC)
        delta = d_ref[0, sl, :]
        u = u_ref[0, sl, :]
        dA_s[sl] = jnp.exp(delta[:, None, :] * A[None, :, :])
        bm = bm_ref[0, sl, :]
        dBu_s[sl] = (delta * u)[:, None, :] * bm[:, :, None]
        return 0

    jax.lax.fori_loop(0, L // _PC, pre, 0)

    def step(t, h):
        h = dA_s[t] * h + dBu_s[t]
        dA_s[t] = h
        return h

    jax.lax.fori_loop(0, L, step, jnp.zeros((NS, TDI), jnp.float32))

    dp = dp_ref[...]

    def post(i, _):
        sl = pl.ds(i * _PC, _PC)
        hist = dA_s[sl]
        cm = cm_ref[0, sl, :]
        y = jnp.sum(hist * cm[:, :, None], axis=1)
        y = y + u_ref[0, sl, :] * dp
        o_ref[0, sl, :] = y * _silu(z_ref[0, sl, :])
        return 0

    jax.lax.fori_loop(0, L // _PC, post, 0)


def _scan(xc, delta, bm, cm, z, a_logT, dp2):
    grid = (B * NB,)
    return pl.pallas_call(
        _scan_kernel,
        out_shape=jax.ShapeDtypeStruct((B, L, DI), jnp.float32),
        grid=grid,
        in_specs=[
            pl.BlockSpec((1, L, TDI), lambda i: (i // NB, 0, i % NB)),
            pl.BlockSpec((1, L, TDI), lambda i: (i // NB, 0, i % NB)),
            pl.BlockSpec((1, L, NS), lambda i: (i // NB, 0, 0)),
            pl.BlockSpec((1, L, NS), lambda i: (i // NB, 0, 0)),
            pl.BlockSpec((1, L, TDI), lambda i: (i // NB, 0, i % NB)),
            pl.BlockSpec((NS, TDI), lambda i: (0, i % NB)),
            pl.BlockSpec((1, TDI), lambda i: (0, i % NB)),
        ],
        out_specs=pl.BlockSpec((1, L, TDI), lambda i: (i // NB, 0, i % NB)),
        scratch_shapes=[
            pltpu.VMEM((L, NS, TDI), jnp.float32),
            pltpu.VMEM((L, NS, TDI), jnp.float32),
        ],
        compiler_params=pltpu.CompilerParams(
            dimension_semantics=("parallel",),
            vmem_limit_bytes=52 * 1024 * 1024,
        ),
        name="mamba_scan",
        interpret=_INTERPRET,
    )(xc, delta, bm, cm, z, a_logT, dp2)


def _down_kernel(yz_ref, w_ref, x_ref, o_ref):
    o_ref[...] = x_ref[...] + jnp.dot(
        yz_ref[...], w_ref[...], preferred_element_type=jnp.float32)


def _down(yz2, wblk, x2):
    M = B * L
    return pl.pallas_call(
        _down_kernel,
        out_shape=jax.ShapeDtypeStruct((M, D), jnp.float32),
        grid=(M // _MT,),
        in_specs=[
            pl.BlockSpec((_MT, DI), lambda i: (i, 0)),
            pl.BlockSpec((DI, D), lambda i: (0, 0)),
            pl.BlockSpec((_MT, D), lambda i: (i, 0)),
        ],
        out_specs=pl.BlockSpec((_MT, D), lambda i: (i, 0)),
        compiler_params=pltpu.CompilerParams(
            dimension_semantics=("parallel",),
            vmem_limit_bytes=48 * 1024 * 1024,
        ),
        name="mamba_down",
        interpret=_INTERPRET,
    )(yz2, wblk, x2)


def _out_kernel(h_ref, w_ref, b_ref, o_ref):
    o_ref[...] = jnp.dot(
        h_ref[...], w_ref[...], preferred_element_type=jnp.float32) + b_ref[...]


def _out_proj(h2, wout, bout2):
    M = B * L
    OD = wout.shape[1]
    return pl.pallas_call(
        _out_kernel,
        out_shape=jax.ShapeDtypeStruct((M, OD), jnp.float32),
        grid=(M // _MT,),
        in_specs=[
            pl.BlockSpec((_MT, D), lambda i: (i, 0)),
            pl.BlockSpec((D, OD), lambda i: (0, 0)),
            pl.BlockSpec((1, OD), lambda i: (0, 0)),
        ],
        out_specs=pl.BlockSpec((_MT, OD), lambda i: (i, 0)),
        compiler_params=pltpu.CompilerParams(
            dimension_semantics=("parallel",),
            vmem_limit_bytes=48 * 1024 * 1024,
        ),
        name="mamba_out",
        interpret=_INTERPRET,
    )(h2, wout, bout2)


def kernel(x, norm_w, Win, conv_w, conv_b, Wx, Wdt, bdt, A_log, Dp, Wblk,
           Wout, bout):
    h = x
    for l in range(Win.shape[0]):
        nw = norm_w[l][None, :]
        cwT = conv_w[l].T
        cb = conv_b[l][None, :]
        bdt2 = bdt[l][None, :]
        a_logT = A_log[l].T
        dp2 = Dp[l][None, :]

        o = _proj(h, nw, Win[l], cwT, cb)
        xc = o[:, 0]
        z = o[:, 1]
        xc2 = xc.reshape(B * L, DI)
        delta2, bm2, cm2 = _dbc(xc2, Wx[l], Wdt[l], bdt2)
        yz = _scan(xc, delta2.reshape(B, L, DI), bm2.reshape(B, L, NS),
                   cm2.reshape(B, L, NS), z, a_logT, dp2)
        h2 = _down(yz.reshape(B * L, DI), Wblk[l], h.reshape(B * L, D))
        h = h2.reshape(B, L, D)

    out2 = _out_proj(h.reshape(B * L, D), Wout, bout[None, :])
    return out2.reshape(B, L, Wout.shape[1])


# speedup vs baseline: 14.0638x; 1.2349x over previous
import jax
import jax.numpy as jnp
from jax.experimental import pallas as pl
from jax.experimental.pallas import tpu as pltpu

_INTERPRET = False

B, L, D = 2, 1024, 1024
DI = 2048
NS = 16
DTR = 64
KC = 4
EPS = 1e-5

TDI = 256
NB = DI // TDI
LT = 256
NLT = L // LT


def _silu(v):
    return v * jax.nn.sigmoid(v)


def _proj_kernel(x_ref, nw_ref, win_ref, cw_ref, cb_ref, o_ref, carry):
    j = pl.program_id(1)
    lt = pl.program_id(2)
    x = x_ref[0]
    ssq = jnp.mean(x * x, axis=-1, keepdims=True)
    xn = x * jax.lax.rsqrt(ssq + EPS) * nw_ref[...]
    xz = jnp.dot(xn, win_ref[...], preferred_element_type=jnp.float32)

    @pl.when(lt == 0)
    def _():
        carry[...] = jnp.zeros((KC - 1, DI), jnp.float32)

    w = cw_ref[...]
    xfull = jnp.concatenate([carry[...], xz], axis=0)
    carry[...] = xz[LT - (KC - 1):, :]
    acc = cb_ref[...]
    for k in range(KC):
        acc = acc + xfull[k:k + LT] * w[k][None, :]
    o_ref[0, 0] = jnp.where(j == 0, _silu(acc), xz)


def _proj(x, nw, win, cwT, cb):
    return pl.pallas_call(
        _proj_kernel,
        out_shape=jax.ShapeDtypeStruct((B, 2, L, DI), jnp.float32),
        grid=(B, 2, NLT),
        in_specs=[
            pl.BlockSpec((1, LT, D), lambda b, j, lt: (b, lt, 0)),
            pl.BlockSpec((1, D), lambda b, j, lt: (0, 0)),
            pl.BlockSpec((D, DI), lambda b, j, lt: (0, j)),
            pl.BlockSpec((KC, DI), lambda b, j, lt: (0, 0)),
            pl.BlockSpec((1, DI), lambda b, j, lt: (0, 0)),
        ],
        out_specs=pl.BlockSpec((1, 1, LT, DI), lambda b, j, lt: (b, j, lt, 0)),
        scratch_shapes=[pltpu.VMEM((KC - 1, DI), jnp.float32)],
        compiler_params=pltpu.CompilerParams(
            dimension_semantics=("parallel", "arbitrary", "arbitrary"),
            vmem_limit_bytes=48 * 1024 * 1024,
        ),
        name="mamba_proj",
        interpret=_INTERPRET,
    )(x, nw, win, cwT, cb)


_MT = 256
_NMT = B * L // _MT


def _dbc_kernel(xc_ref, wx_ref, wdt_ref, bdt_ref, d_ref, b_ref, c_ref):
    xc = xc_ref[0, 0]
    dbc = jnp.dot(xc, wx_ref[...], preferred_element_type=jnp.float32)
    dt = dbc[:, :DTR]
    d_ref[0] = jax.nn.softplus(
        jnp.dot(dt, wdt_ref[...], preferred_element_type=jnp.float32)
        + bdt_ref[...])
    b_ref[0] = dbc[:, DTR:DTR + NS]
    c_ref[0] = dbc[:, DTR + NS:DTR + 2 * NS]


def _dbc(o, wx, wdt, bdt2):
    nlt = L // _MT
    return pl.pallas_call(
        _dbc_kernel,
        out_shape=(
            jax.ShapeDtypeStruct((B, L, DI), jnp.float32),
            jax.ShapeDtypeStruct((B, L, NS), jnp.float32),
            jax.ShapeDtypeStruct((B, L, NS), jnp.float32),
        ),
        grid=(_NMT,),
        in_specs=[
            pl.BlockSpec((1, 1, _MT, DI), lambda i: (i // nlt, 0, i % nlt, 0)),
            pl.BlockSpec((DI, DTR + 2 * NS), lambda i: (0, 0)),
            pl.BlockSpec((DTR, DI), lambda i: (0, 0)),
            pl.BlockSpec((1, DI), lambda i: (0, 0)),
        ],
        out_specs=(
            pl.BlockSpec((1, _MT, DI), lambda i: (i // nlt, i % nlt, 0)),
            pl.BlockSpec((1, _MT, NS), lambda i: (i // nlt, i % nlt, 0)),
            pl.BlockSpec((1, _MT, NS), lambda i: (i // nlt, i % nlt, 0)),
        ),
        compiler_params=pltpu.CompilerParams(
            dimension_semantics=("parallel",),
            vmem_limit_bytes=48 * 1024 * 1024,
        ),
        name="mamba_dbc",
        interpret=_INTERPRET,
    )(o, wx, wdt, bdt2)


_PC = 128
L2 = L // 2


def _scan_kernel(u_ref, d_ref, bm_ref, cm_ref, z_ref, at_ref, dp_ref,
                 o_ref, dA_s, dBu_s):
    A = -jnp.exp(at_ref[...])

    def pre(i, _):
        sl = pl.ds(i * _PC, _PC)
        delta = d_ref[0, sl, :]
        u = u_ref[0, 0, sl, :]
        dA = jnp.exp(delta[:, None, :] * A[None, :, :])
        bm = bm_ref[0, sl, :]
        dBu = (delta * u)[:, None, :] * bm[:, :, None]
        a4 = dA.reshape(_PC // 2, 2, NS, TDI)
        b4 = dBu.reshape(_PC // 2, 2, NS, TDI)
        a0 = a4[:, 0]
        a1 = a4[:, 1]
        b0 = b4[:, 0]
        b1 = b4[:, 1]
        sl2 = pl.ds(i * (_PC // 2), _PC // 2)
        dA_s[sl2, 0] = a0
        dA_s[sl2, 1] = a1 * a0
        dBu_s[sl2, 0] = b0
        dBu_s[sl2, 1] = a1 * b0 + b1
        return 0

    jax.lax.fori_loop(0, L // _PC, pre, 0)

    def step(k, h):
        cA = dA_s[k, 1]
        cB = dBu_s[k, 1]
        dBu_s[k, 1] = h
        h = cA * h + cB
        dA_s[k, 1] = h
        return h

    jax.lax.fori_loop(0, L2, step, jnp.zeros((NS, TDI), jnp.float32),
                      unroll=4)

    _C2 = 64

    def expand(i, _):
        sl2 = pl.ds(i * _C2, _C2)
        hm1 = dBu_s[sl2, 1]
        dA_s[sl2, 0] = dA_s[sl2, 0] * hm1 + dBu_s[sl2, 0]
        return 0

    jax.lax.fori_loop(0, L2 // _C2, expand, 0)

    dp = dp_ref[...]

    def post(i, _):
        sl2 = pl.ds(i * _C2, _C2)
        sl = pl.ds(i * 2 * _C2, 2 * _C2)
        hist = dA_s[sl2]
        cm = cm_ref[0, sl, :].reshape(_C2, 2, NS)
        y = jnp.sum(hist * cm[:, :, :, None], axis=2)
        y = y.reshape(2 * _C2, TDI)
        y = y + u_ref[0, 0, sl, :] * dp
        zv = z_ref[0, 0, sl, :]
        o_ref[0, sl, :] = y * _silu(zv)
        return 0

    jax.lax.fori_loop(0, L2 // _C2, post, 0)


def _scan(o, delta, bm, cm, a_logT, dp2):
    grid = (B * NB,)
    return pl.pallas_call(
        _scan_kernel,
        out_shape=jax.ShapeDtypeStruct((B, L, DI), jnp.float32),
        grid=grid,
        in_specs=[
            pl.BlockSpec((1, 1, L, TDI), lambda i: (i // NB, 0, 0, i % NB)),
            pl.BlockSpec((1, L, TDI), lambda i: (i // NB, 0, i % NB)),
            pl.BlockSpec((1, L, NS), lambda i: (i // NB, 0, 0)),
            pl.BlockSpec((1, L, NS), lambda i: (i // NB, 0, 0)),
            pl.BlockSpec((1, 1, L, TDI), lambda i: (i // NB, 1, 0, i % NB)),
            pl.BlockSpec((NS, TDI), lambda i: (0, i % NB)),
            pl.BlockSpec((1, TDI), lambda i: (0, i % NB)),
        ],
        out_specs=pl.BlockSpec((1, L, TDI), lambda i: (i // NB, 0, i % NB)),
        scratch_shapes=[
            pltpu.VMEM((L2, 2, NS, TDI), jnp.float32),
            pltpu.VMEM((L2, 2, NS, TDI), jnp.float32),
        ],
        compiler_params=pltpu.CompilerParams(
            dimension_semantics=("parallel",),
            vmem_limit_bytes=52 * 1024 * 1024,
        ),
        name="mamba_scan",
        interpret=_INTERPRET,
    )(o, delta, bm, cm, o, a_logT, dp2)


def _down_kernel(yz_ref, w_ref, x_ref, o_ref):
    o_ref[0] = x_ref[0] + jnp.dot(
        yz_ref[0], w_ref[...], preferred_element_type=jnp.float32)


def _down(yz, wblk, x):
    nlt = L // _MT
    return pl.pallas_call(
        _down_kernel,
        out_shape=jax.ShapeDtypeStruct((B, L, D), jnp.float32),
        grid=(_NMT,),
        in_specs=[
            pl.BlockSpec((1, _MT, DI), lambda i: (i // nlt, i % nlt, 0)),
            pl.BlockSpec((DI, D), lambda i: (0, 0)),
            pl.BlockSpec((1, _MT, D), lambda i: (i // nlt, i % nlt, 0)),
        ],
        out_specs=pl.BlockSpec((1, _MT, D), lambda i: (i // nlt, i % nlt, 0)),
        compiler_params=pltpu.CompilerParams(
            dimension_semantics=("parallel",),
            vmem_limit_bytes=48 * 1024 * 1024,
        ),
        name="mamba_down",
        interpret=_INTERPRET,
    )(yz, wblk, x)


def _out_kernel(h_ref, w_ref, b_ref, o_ref):
    o_ref[...] = jnp.dot(
        h_ref[...], w_ref[...], preferred_element_type=jnp.float32) + b_ref[...]


def _out_proj(h2, wout, bout2):
    M = B * L
    OD = wout.shape[1]
    return pl.pallas_call(
        _out_kernel,
        out_shape=jax.ShapeDtypeStruct((M, OD), jnp.float32),
        grid=(M // _MT,),
        in_specs=[
            pl.BlockSpec((_MT, D), lambda i: (i, 0)),
            pl.BlockSpec((D, OD), lambda i: (0, 0)),
            pl.BlockSpec((1, OD), lambda i: (0, 0)),
        ],
        out_specs=pl.BlockSpec((_MT, OD), lambda i: (i, 0)),
        compiler_params=pltpu.CompilerParams(
            dimension_semantics=("parallel",),
            vmem_limit_bytes=48 * 1024 * 1024,
        ),
        name="mamba_out",
        interpret=_INTERPRET,
    )(h2, wout, bout2)


def kernel(x, norm_w, Win, conv_w, conv_b, Wx, Wdt, bdt, A_log, Dp, Wblk,
           Wout, bout):
    h = x
    for l in range(Win.shape[0]):
        nw = norm_w[l][None, :]
        cwT = conv_w[l].T
        cb = conv_b[l][None, :]
        bdt2 = bdt[l][None, :]
        a_logT = A_log[l].T
        dp2 = Dp[l][None, :]

        o = _proj(h, nw, Win[l], cwT, cb)
        delta, bm, cm = _dbc(o, Wx[l], Wdt[l], bdt2)
        yz = _scan(o, delta, bm, cm, a_logT, dp2)
        h = _down(yz, Wblk[l], h)

    out2 = _out_proj(h.reshape(B * L, D), Wout, bout[None, :])
    return out2.reshape(B, L, Wout.shape[1])


# speedup vs baseline: 14.8228x; 1.0540x over previous
import jax
import jax.numpy as jnp
from jax.experimental import pallas as pl
from jax.experimental.pallas import tpu as pltpu

_INTERPRET = False

B, L, D = 2, 1024, 1024
DI = 2048
NS = 16
DTR = 64
KC = 4
EPS = 1e-5

TDI = 256
NB = DI // TDI
LT = 256
NLT = L // LT


def _silu(v):
    return v * jax.nn.sigmoid(v)


def _proj_kernel(x_ref, nw_ref, win_ref, cw_ref, cb_ref, o_ref, carry):
    j = pl.program_id(1)
    lt = pl.program_id(2)
    x = x_ref[0]
    ssq = jnp.mean(x * x, axis=-1, keepdims=True)
    xn = x * jax.lax.rsqrt(ssq + EPS) * nw_ref[0]
    xz = jnp.dot(xn, win_ref[0], preferred_element_type=jnp.float32)

    @pl.when(lt == 0)
    def _():
        carry[...] = jnp.zeros((KC - 1, DI), jnp.float32)

    w = cw_ref[0]
    xfull = jnp.concatenate([carry[...], xz], axis=0)
    carry[...] = xz[LT - (KC - 1):, :]
    acc = cb_ref[0]
    for k in range(KC):
        acc = acc + xfull[k:k + LT] * w[k][None, :]
    o_ref[0, 0] = jnp.where(j == 0, _silu(acc), xz)


def _proj(l, x, nw, win, cwT, cb):
    return pl.pallas_call(
        _proj_kernel,
        out_shape=jax.ShapeDtypeStruct((B, 2, L, DI), jnp.float32),
        grid=(B, 2, NLT),
        in_specs=[
            pl.BlockSpec((1, LT, D), lambda b, j, lt: (b, lt, 0)),
            pl.BlockSpec((1, 1, D), lambda b, j, lt: (l, 0, 0)),
            pl.BlockSpec((1, D, DI), lambda b, j, lt: (l, 0, j)),
            pl.BlockSpec((1, KC, DI), lambda b, j, lt: (l, 0, 0)),
            pl.BlockSpec((1, 1, DI), lambda b, j, lt: (l, 0, 0)),
        ],
        out_specs=pl.BlockSpec((1, 1, LT, DI), lambda b, j, lt: (b, j, lt, 0)),
        scratch_shapes=[pltpu.VMEM((KC - 1, DI), jnp.float32)],
        compiler_params=pltpu.CompilerParams(
            dimension_semantics=("parallel", "arbitrary", "arbitrary"),
            vmem_limit_bytes=48 * 1024 * 1024,
        ),
        name="mamba_proj",
        interpret=_INTERPRET,
    )(x, nw, win, cwT, cb)


_MT = 256
_NMT = B * L // _MT


def _dbc_kernel(xc_ref, wx_ref, wdt_ref, bdt_ref, d_ref, b_ref, c_ref):
    xc = xc_ref[0, 0]
    dbc = jnp.dot(xc, wx_ref[0], preferred_element_type=jnp.float32)
    dt = dbc[:, :DTR]
    d_ref[0] = jax.nn.softplus(
        jnp.dot(dt, wdt_ref[0], preferred_element_type=jnp.float32)
        + bdt_ref[0])
    b_ref[0] = dbc[:, DTR:DTR + NS]
    c_ref[0] = dbc[:, DTR + NS:DTR + 2 * NS]


def _dbc(l, o, wx, wdt, bdt2):
    nlt = L // _MT
    return pl.pallas_call(
        _dbc_kernel,
        out_shape=(
            jax.ShapeDtypeStruct((B, L, DI), jnp.float32),
            jax.ShapeDtypeStruct((B, L, NS), jnp.float32),
            jax.ShapeDtypeStruct((B, L, NS), jnp.float32),
        ),
        grid=(_NMT,),
        in_specs=[
            pl.BlockSpec((1, 1, _MT, DI), lambda i: (i // nlt, 0, i % nlt, 0)),
            pl.BlockSpec((1, DI, DTR + 2 * NS), lambda i: (l, 0, 0)),
            pl.BlockSpec((1, DTR, DI), lambda i: (l, 0, 0)),
            pl.BlockSpec((1, 1, DI), lambda i: (l, 0, 0)),
        ],
        out_specs=(
            pl.BlockSpec((1, _MT, DI), lambda i: (i // nlt, i % nlt, 0)),
            pl.BlockSpec((1, _MT, NS), lambda i: (i // nlt, i % nlt, 0)),
            pl.BlockSpec((1, _MT, NS), lambda i: (i // nlt, i % nlt, 0)),
        ),
        compiler_params=pltpu.CompilerParams(
            dimension_semantics=("parallel",),
            vmem_limit_bytes=48 * 1024 * 1024,
        ),
        name="mamba_dbc",
        interpret=_INTERPRET,
    )(o, wx, wdt, bdt2)


_PC = 128
L2 = L // 2


def _scan_kernel(u_ref, d_ref, bm_ref, cm_ref, z_ref, at_ref, dp_ref,
                 o_ref, dA_s, dBu_s):
    A = -jnp.exp(at_ref[0])

    def pre(i, _):
        sl = pl.ds(i * _PC, _PC)
        delta = d_ref[0, sl, :]
        u = u_ref[0, 0, sl, :]
        dA = jnp.exp(delta[:, None, :] * A[None, :, :])
        bm = bm_ref[0, sl, :]
        dBu = (delta * u)[:, None, :] * bm[:, :, None]
        a4 = dA.reshape(_PC // 2, 2, NS, TDI)
        b4 = dBu.reshape(_PC // 2, 2, NS, TDI)
        a0 = a4[:, 0]
        a1 = a4[:, 1]
        b0 = b4[:, 0]
        b1 = b4[:, 1]
        sl2 = pl.ds(i * (_PC // 2), _PC // 2)
        dA_s[sl2, 0] = a0
        dA_s[sl2, 1] = a1 * a0
        dBu_s[sl2, 0] = b0
        dBu_s[sl2, 1] = a1 * b0 + b1
        return 0

    jax.lax.fori_loop(0, L // _PC, pre, 0)

    def step(k, h):
        cA = dA_s[k, 1]
        cB = dBu_s[k, 1]
        dBu_s[k, 1] = h
        h = cA * h + cB
        dA_s[k, 1] = h
        return h

    jax.lax.fori_loop(0, L2, step, jnp.zeros((NS, TDI), jnp.float32),
                      unroll=4)

    _C2 = 64

    def expand(i, _):
        sl2 = pl.ds(i * _C2, _C2)
        hm1 = dBu_s[sl2, 1]
        dA_s[sl2, 0] = dA_s[sl2, 0] * hm1 + dBu_s[sl2, 0]
        return 0

    jax.lax.fori_loop(0, L2 // _C2, expand, 0)

    dp = dp_ref[0]

    def post(i, _):
        sl2 = pl.ds(i * _C2, _C2)
        sl = pl.ds(i * 2 * _C2, 2 * _C2)
        hist = dA_s[sl2]
        cm = cm_ref[0, sl, :].reshape(_C2, 2, NS)
        y = jnp.sum(hist * cm[:, :, :, None], axis=2)
        y = y.reshape(2 * _C2, TDI)
        y = y + u_ref[0, 0, sl, :] * dp
        zv = z_ref[0, 0, sl, :]
        o_ref[0, sl, :] = y * _silu(zv)
        return 0

    jax.lax.fori_loop(0, L2 // _C2, post, 0)


def _scan(l, o, delta, bm, cm, a_logT, dp2):
    grid = (B * NB,)
    return pl.pallas_call(
        _scan_kernel,
        out_shape=jax.ShapeDtypeStruct((B, L, DI), jnp.float32),
        grid=grid,
        in_specs=[
            pl.BlockSpec((1, 1, L, TDI), lambda i: (i // NB, 0, 0, i % NB)),
            pl.BlockSpec((1, L, TDI), lambda i: (i // NB, 0, i % NB)),
            pl.BlockSpec((1, L, NS), lambda i: (i // NB, 0, 0)),
            pl.BlockSpec((1, L, NS), lambda i: (i // NB, 0, 0)),
            pl.BlockSpec((1, 1, L, TDI), lambda i: (i // NB, 1, 0, i % NB)),
            pl.BlockSpec((1, NS, TDI), lambda i: (l, 0, i % NB)),
            pl.BlockSpec((1, 1, TDI), lambda i: (l, 0, i % NB)),
        ],
        out_specs=pl.BlockSpec((1, L, TDI), lambda i: (i // NB, 0, i % NB)),
        scratch_shapes=[
            pltpu.VMEM((L2, 2, NS, TDI), jnp.float32),
            pltpu.VMEM((L2, 2, NS, TDI), jnp.float32),
        ],
        compiler_params=pltpu.CompilerParams(
            dimension_semantics=("parallel",),
            vmem_limit_bytes=52 * 1024 * 1024,
        ),
        name="mamba_scan",
        interpret=_INTERPRET,
    )(o, delta, bm, cm, o, a_logT, dp2)


def _down_kernel(yz_ref, w_ref, x_ref, o_ref):
    o_ref[0] = x_ref[0] + jnp.dot(
        yz_ref[0], w_ref[0], preferred_element_type=jnp.float32)


def _down(l, yz, wblk, x):
    nlt = L // _MT
    return pl.pallas_call(
        _down_kernel,
        out_shape=jax.ShapeDtypeStruct((B, L, D), jnp.float32),
        grid=(_NMT,),
        in_specs=[
            pl.BlockSpec((1, _MT, DI), lambda i: (i // nlt, i % nlt, 0)),
            pl.BlockSpec((1, DI, D), lambda i: (l, 0, 0)),
            pl.BlockSpec((1, _MT, D), lambda i: (i // nlt, i % nlt, 0)),
        ],
        out_specs=pl.BlockSpec((1, _MT, D), lambda i: (i // nlt, i % nlt, 0)),
        compiler_params=pltpu.CompilerParams(
            dimension_semantics=("parallel",),
            vmem_limit_bytes=48 * 1024 * 1024,
        ),
        name="mamba_down",
        interpret=_INTERPRET,
    )(yz, wblk, x)


def _out_kernel(h_ref, w_ref, b_ref, o_ref):
    o_ref[...] = jnp.dot(
        h_ref[...], w_ref[...], preferred_element_type=jnp.float32) + b_ref[...]


def _out_proj(h2, wout, bout2):
    M = B * L
    OD = wout.shape[1]
    return pl.pallas_call(
        _out_kernel,
        out_shape=jax.ShapeDtypeStruct((M, OD), jnp.float32),
        grid=(M // _MT,),
        in_specs=[
            pl.BlockSpec((_MT, D), lambda i: (i, 0)),
            pl.BlockSpec((D, OD), lambda i: (0, 0)),
            pl.BlockSpec((1, OD), lambda i: (0, 0)),
        ],
        out_specs=pl.BlockSpec((_MT, OD), lambda i: (i, 0)),
        compiler_params=pltpu.CompilerParams(
            dimension_semantics=("parallel",),
            vmem_limit_bytes=48 * 1024 * 1024,
        ),
        name="mamba_out",
        interpret=_INTERPRET,
    )(h2, wout, bout2)


def kernel(x, norm_w, Win, conv_w, conv_b, Wx, Wdt, bdt, A_log, Dp, Wblk,
           Wout, bout):
    h = x
    nw = norm_w[:, None, :]
    cwT = conv_w.transpose(0, 2, 1)
    cb = conv_b[:, None, :]
    bdt2 = bdt[:, None, :]
    a_logT = A_log.transpose(0, 2, 1)
    dp2 = Dp[:, None, :]
    for l in range(Win.shape[0]):
        o = _proj(l, h, nw, Win, cwT, cb)
        delta, bm, cm = _dbc(l, o, Wx, Wdt, bdt2)
        yz = _scan(l, o, delta, bm, cm, a_logT, dp2)
        h = _down(l, yz, Wblk, h)

    out2 = _out_proj(h.reshape(B * L, D), Wout, bout[None, :])
    return out2.reshape(B, L, Wout.shape[1])
